# baseline (speedup 1.0000x reference)
import functools

import jax
import jax.numpy as jnp
from jax import lax
from jax.experimental import pallas as pl
from jax.experimental.pallas import tpu as pltpu

B = 1024
L = 20
DIM = 100
NUM_TOTAL = 100000
ALPHA = 0.2

BB = 8
BBL = BB * L
NEG = -9e15

NW = 32
DPAD = 128
ROWS_TOTAL = 3 * B * L
ROWS_PER_W = ROWS_TOTAL // NW
CHUNK = 128
NCHUNK = ROWS_PER_W // CHUNK


def _make_sc_gather():
    from jax.experimental.pallas import tpu_sc as plsc

    mesh = plsc.VectorSubcoreMesh(core_axis_name="c", subcore_axis_name="s")

    @functools.partial(
        pl.kernel,
        mesh=mesh,
        out_type=jax.ShapeDtypeStruct((ROWS_TOTAL, DPAD), jnp.float32),
        scratch_types=[
            pltpu.VMEM((CHUNK,), jnp.int32),
            pltpu.VMEM((CHUNK, DPAD), jnp.float32),
            pltpu.SemaphoreType.DMA,
        ],
    )
    def gather_kernel(table_hbm, idx_hbm, out_hbm, idx_v, rows_v, sem):
        wid = lax.axis_index("s") * 2 + lax.axis_index("c")
        base = wid * ROWS_PER_W

        def chunk_body(i, carry):
            off = base + i * CHUNK
            pltpu.sync_copy(idx_hbm.at[pl.ds(off, CHUNK)], idx_v)
            pltpu.async_copy(table_hbm.at[idx_v], rows_v, sem).wait()
            pltpu.sync_copy(rows_v, out_hbm.at[pl.ds(off, CHUNK)])
            return carry

        lax.fori_loop(0, NCHUNK, chunk_body, 0)

    return gather_kernel


def _leaky(x):
    return jnp.where(x >= 0, x, ALPHA * x)


def _local_agg_block(X, adj_r, a_t, same, neg_row):
    adj_t = jnp.concatenate([adj_r] * BB, axis=1)
    logits = neg_row
    dims = (((1,), (1,)), ((), ()))
    for k in range(4):
        Ek = lax.dot_general(X * a_t[k][None, :], X, dims,
                             preferred_element_type=jnp.float32)
        logits = jnp.where(same & (adj_t == k + 1), _leaky(Ek), logits)
    m = jnp.max(logits, axis=1, keepdims=True)
    p = jnp.exp(logits - m)
    alpha = p / jnp.sum(p, axis=1, keepdims=True)
    return jnp.dot(alpha, X, preferred_element_type=jnp.float32)


def _tc_body(h1_ref, adj_ref, h2_ref, aid_ref, hm_ref, tadj_ref,
             la1_ref, mix_ref, Wei_ref, bei_ref, Weo_ref, beo_ref,
             wr_in_ref, wr_out_ref, wi_in_ref, wi_out_ref, wn_in_ref, wn_out_ref,
             whr_ref, whi_ref, whn_ref,
             br_ref, bi_ref, bn_ref, bhr_ref, bhi_ref, bhn_ref,
             biah_ref, boah_ref,
             o1_ref, o2_ref, om_ref):
    r = lax.broadcasted_iota(jnp.int32, (BBL, BBL), 0)
    c = lax.broadcasted_iota(jnp.int32, (BBL, BBL), 1)
    same = (r // L) == (c // L)
    neg_row = jnp.where(same, jnp.float32(NEG), -jnp.inf)
    samef = same.astype(jnp.float32)

    dims = (((1,), (1,)), ((), ()))

    X1 = h1_ref[...][:, :DIM]
    o1_ref[...] = _local_agg_block(X1, adj_ref[...], la1_ref[...], same, neg_row)

    Xm = hm_ref[...][:, :DIM]
    om_ref[...] = _local_agg_block(Xm, tadj_ref[...], mix_ref[...], same, neg_row)

    X2 = h2_ref[...][:, :DIM]
    aid = aid_ref[...]
    Ain = jnp.concatenate([aid[:, :L]] * BB, axis=1) * samef
    Aout = jnp.concatenate([aid[:, L:]] * BB, axis=1) * samef

    Vi = lax.dot_general(X2, Wei_ref[...], dims,
                         preferred_element_type=jnp.float32) + bei_ref[...]
    Vo = lax.dot_general(X2, Weo_ref[...], dims,
                         preferred_element_type=jnp.float32) + beo_ref[...]
    input_in = jnp.dot(Ain, Vi, preferred_element_type=jnp.float32) + biah_ref[...]
    input_out = jnp.dot(Aout, Vo, preferred_element_type=jnp.float32) + boah_ref[...]

    def mm(x, w_ref):
        return lax.dot_general(x, w_ref[...], dims,
                               preferred_element_type=jnp.float32)

    gi_r = mm(input_in, wr_in_ref) + mm(input_out, wr_out_ref) + br_ref[...]
    gi_i = mm(input_in, wi_in_ref) + mm(input_out, wi_out_ref) + bi_ref[...]
    gi_n = mm(input_in, wn_in_ref) + mm(input_out, wn_out_ref) + bn_ref[...]
    gh_r = mm(X2, whr_ref) + bhr_ref[...]
    gh_i = mm(X2, whi_ref) + bhi_ref[...]
    gh_n = mm(X2, whn_ref) + bhn_ref[...]

    resetgate = jax.nn.sigmoid(gi_r + gh_r)
    inputgate = jax.nn.sigmoid(gi_i + gh_i)
    newgate = jnp.tanh(gi_n + resetgate * gh_n)
    o2_ref[...] = newgate + inputgate * (newgate - X2)


def _tc_call(h1r, adj_r, h2r, aid_r, hmr, tadj_r, weights, interpret=False):
    nblk = B // BB
    row_spec = lambda w, dt: pl.BlockSpec((BBL, w), lambda i: (i, 0))
    full = lambda a: pl.BlockSpec(a.shape, lambda i: (0,) * a.ndim)

    hwid = h1r.shape[1]
    in_specs = [
        pl.BlockSpec((BBL, hwid), lambda i: (i, 0)),
        pl.BlockSpec((BBL, L), lambda i: (i, 0)),
        pl.BlockSpec((BBL, hwid), lambda i: (i, 0)),
        pl.BlockSpec((BBL, 2 * L), lambda i: (i, 0)),
        pl.BlockSpec((BBL, hwid), lambda i: (i, 0)),
        pl.BlockSpec((BBL, L), lambda i: (i, 0)),
    ] + [full(w) for w in weights]

    out_specs = [pl.BlockSpec((BBL, DIM), lambda i: (i, 0))] * 3
    out_shape = [jax.ShapeDtypeStruct((B * L, DIM), jnp.float32)] * 3

    return pl.pallas_call(
        _tc_body,
        grid=(nblk,),
        in_specs=in_specs,
        out_specs=out_specs,
        out_shape=out_shape,
        interpret=interpret,
    )(h1r, adj_r, h2r, aid_r, hmr, tadj_r, *weights)


def _prep_weights(la1_a, mix_a, Wei, bei, Weo, beo, w_ih, w_hh, b_ih, b_hh,
                  b_iah, b_oah):
    w_r, w_i, w_n = w_ih[:DIM], w_ih[DIM:2 * DIM], w_ih[2 * DIM:]
    wh_r, wh_i, wh_n = w_hh[:DIM], w_hh[DIM:2 * DIM], w_hh[2 * DIM:]
    row = lambda v: v.reshape(1, -1)
    return [
        la1_a.T, mix_a.T,
        Wei, row(bei), Weo, row(beo),
        w_r[:, :DIM], w_r[:, DIM:],
        w_i[:, :DIM], w_i[:, DIM:],
        w_n[:, :DIM], w_n[:, DIM:],
        wh_r, wh_i, wh_n,
        row(b_ih[:DIM]), row(b_ih[DIM:2 * DIM]), row(b_ih[2 * DIM:]),
        row(b_hh[:DIM]), row(b_hh[DIM:2 * DIM]), row(b_hh[2 * DIM:]),
        row(b_iah), row(b_oah),
    ]


def kernel(inputs, adj, mask_item, item, items_ID, adj_ID, total_items,
           total_adj, embedding, la1_a, mix_a, Wei, bei, Weo, beo,
           w_ih, w_hh, b_ih, b_hh, b_iah, b_oah):
    idx_all = jnp.concatenate([
        inputs.reshape(-1), items_ID.reshape(-1), total_items.reshape(-1)
    ]).astype(jnp.int32)

    emb_p = jnp.pad(embedding, ((0, 0), (0, DPAD - DIM)))
    rows = _make_sc_gather()(emb_p, idx_all)
    h1r = rows[:B * L]
    h2r = rows[B * L:2 * B * L]
    hmr = rows[2 * B * L:]

    weights = _prep_weights(la1_a, mix_a, Wei, bei, Weo, beo, w_ih, w_hh,
                            b_ih, b_hh, b_iah, b_oah)

    o1, o2, om = _tc_call(
        h1r, adj.reshape(B * L, L), h2r, adj_ID.reshape(B * L, 2 * L),
        hmr, total_adj.reshape(B * L, L), weights)

    shp = (B, L, DIM)
    return (o1.reshape(shp), o2.reshape(shp), om.reshape(shp))


# speedup vs baseline: 1.2503x; 1.2503x over previous
import functools

import jax
import jax.numpy as jnp
from jax import lax
from jax.experimental import pallas as pl
from jax.experimental.pallas import tpu as pltpu

B = 1024
L = 20
DIM = 100
NUM_TOTAL = 100000
ALPHA = 0.2

BB = 8
BBL = BB * L
NEG = -9e15

NW = 32
DPAD = 128
ROWS_TOTAL = 3 * B * L
ROWS_PER_W = ROWS_TOTAL // NW
CHUNK = 128
NCHUNK = ROWS_PER_W // CHUNK


SEG = B * L
SEG_PER_W = SEG // NW
NCHUNK_SEG = SEG_PER_W // CHUNK


def _make_sc_gather():
    from jax.experimental.pallas import tpu_sc as plsc

    mesh = plsc.VectorSubcoreMesh(core_axis_name="c", subcore_axis_name="s")

    @functools.partial(
        pl.kernel,
        mesh=mesh,
        out_type=jax.ShapeDtypeStruct((ROWS_TOTAL, DPAD), jnp.float32),
        scratch_types=[
            pltpu.VMEM((CHUNK,), jnp.int32),
            pltpu.VMEM((CHUNK, DPAD), jnp.float32),
            pltpu.SemaphoreType.DMA,
        ],
    )
    def gather_kernel(table_hbm, i0_hbm, i1_hbm, i2_hbm, out_hbm,
                      idx_v, rows_v, sem):
        wid = lax.axis_index("s") * 2 + lax.axis_index("c")
        base = wid * SEG_PER_W

        for seg, idx_hbm in enumerate((i0_hbm, i1_hbm, i2_hbm)):
            def chunk_body(i, carry, idx_hbm=idx_hbm, seg=seg):
                off = base + i * CHUNK
                pltpu.sync_copy(idx_hbm.at[pl.ds(off, CHUNK)], idx_v)
                pltpu.async_copy(table_hbm.at[idx_v], rows_v, sem).wait()
                pltpu.sync_copy(rows_v, out_hbm.at[pl.ds(seg * SEG + off, CHUNK)])
                return carry

            lax.fori_loop(0, NCHUNK_SEG, chunk_body, 0)

    return gather_kernel


PAD_ROWS = 2000


def _pad_body(src_ref, dst_ref):
    dst_ref[...] = jnp.concatenate(
        [src_ref[...], jnp.zeros((PAD_ROWS, DPAD - DIM), jnp.float32)], axis=1)


def _pad_table(emb):
    return pl.pallas_call(
        _pad_body,
        grid=(NUM_TOTAL // PAD_ROWS,),
        in_specs=[pl.BlockSpec((PAD_ROWS, DIM), lambda i: (i, 0))],
        out_specs=pl.BlockSpec((PAD_ROWS, DPAD), lambda i: (i, 0)),
        out_shape=jax.ShapeDtypeStruct((NUM_TOTAL, DPAD), jnp.float32),
    )(emb)


def _leaky(x):
    return jnp.where(x >= 0, x, ALPHA * x)


def _local_agg_block(X, adj_r, a_t, same, neg_row):
    adj_t = jnp.concatenate([adj_r] * BB, axis=1)
    logits = neg_row
    dims = (((1,), (1,)), ((), ()))
    for k in range(4):
        Ek = lax.dot_general(X * a_t[k][None, :], X, dims,
                             preferred_element_type=jnp.float32)
        logits = jnp.where(same & (adj_t == k + 1), _leaky(Ek), logits)
    m = jnp.max(logits, axis=1, keepdims=True)
    p = jnp.exp(logits - m)
    alpha = p / jnp.sum(p, axis=1, keepdims=True)
    return jnp.dot(alpha, X, preferred_element_type=jnp.float32)


def _tc_body(h1_ref, adj_ref, h2_ref, aid_ref, hm_ref, tadj_ref,
             la1_ref, mix_ref, Wei_ref, bei_ref, Weo_ref, beo_ref,
             wr_in_ref, wr_out_ref, wi_in_ref, wi_out_ref, wn_in_ref, wn_out_ref,
             whr_ref, whi_ref, whn_ref,
             br_ref, bi_ref, bn_ref, bhr_ref, bhi_ref, bhn_ref,
             biah_ref, boah_ref,
             o1_ref, o2_ref, om_ref):
    r = lax.broadcasted_iota(jnp.int32, (BBL, BBL), 0)
    c = lax.broadcasted_iota(jnp.int32, (BBL, BBL), 1)
    same = (r // L) == (c // L)
    neg_row = jnp.where(same, jnp.float32(NEG), -jnp.inf)
    samef = same.astype(jnp.float32)

    dims = (((1,), (1,)), ((), ()))

    X1 = h1_ref[...][:, :DIM]
    o1_ref[...] = _local_agg_block(X1, adj_ref[...], la1_ref[...], same, neg_row)

    Xm = hm_ref[...][:, :DIM]
    om_ref[...] = _local_agg_block(Xm, tadj_ref[...], mix_ref[...], same, neg_row)

    X2 = h2_ref[...][:, :DIM]
    aid = aid_ref[...]
    Ain = jnp.concatenate([aid[:, :L]] * BB, axis=1) * samef
    Aout = jnp.concatenate([aid[:, L:]] * BB, axis=1) * samef

    Vi = lax.dot_general(X2, Wei_ref[...], dims,
                         preferred_element_type=jnp.float32) + bei_ref[...]
    Vo = lax.dot_general(X2, Weo_ref[...], dims,
                         preferred_element_type=jnp.float32) + beo_ref[...]
    input_in = jnp.dot(Ain, Vi, preferred_element_type=jnp.float32) + biah_ref[...]
    input_out = jnp.dot(Aout, Vo, preferred_element_type=jnp.float32) + boah_ref[...]

    def mm(x, w_ref):
        return lax.dot_general(x, w_ref[...], dims,
                               preferred_element_type=jnp.float32)

    gi_r = mm(input_in, wr_in_ref) + mm(input_out, wr_out_ref) + br_ref[...]
    gi_i = mm(input_in, wi_in_ref) + mm(input_out, wi_out_ref) + bi_ref[...]
    gi_n = mm(input_in, wn_in_ref) + mm(input_out, wn_out_ref) + bn_ref[...]
    gh_r = mm(X2, whr_ref) + bhr_ref[...]
    gh_i = mm(X2, whi_ref) + bhi_ref[...]
    gh_n = mm(X2, whn_ref) + bhn_ref[...]

    resetgate = jax.nn.sigmoid(gi_r + gh_r)
    inputgate = jax.nn.sigmoid(gi_i + gh_i)
    newgate = jnp.tanh(gi_n + resetgate * gh_n)
    o2_ref[...] = newgate + inputgate * (newgate - X2)


def _tc_call(rows, adj_r, aid_r, tadj_r, weights, interpret=False):
    nblk = B // BB
    full = lambda a: pl.BlockSpec(a.shape, lambda i: (0,) * a.ndim)

    if isinstance(rows, tuple):
        h1r, h2r, hmr = rows
        hwid = h1r.shape[1]
        s1 = pl.BlockSpec((BBL, hwid), lambda i: (i, 0))
        s2 = pl.BlockSpec((BBL, hwid), lambda i: (i, 0))
        s3 = pl.BlockSpec((BBL, hwid), lambda i: (i, 0))
    else:
        h1r = h2r = hmr = rows
        hwid = rows.shape[1]
        off = SEG // BBL
        s1 = pl.BlockSpec((BBL, hwid), lambda i: (i, 0))
        s2 = pl.BlockSpec((BBL, hwid), lambda i: (i + off, 0))
        s3 = pl.BlockSpec((BBL, hwid), lambda i: (i + 2 * off, 0))

    in_specs = [
        s1,
        pl.BlockSpec((BBL, L), lambda i: (i, 0)),
        s2,
        pl.BlockSpec((BBL, 2 * L), lambda i: (i, 0)),
        s3,
        pl.BlockSpec((BBL, L), lambda i: (i, 0)),
    ] + [full(w) for w in weights]

    out_specs = [pl.BlockSpec((BBL, DIM), lambda i: (i, 0))] * 3
    out_shape = [jax.ShapeDtypeStruct((B * L, DIM), jnp.float32)] * 3

    return pl.pallas_call(
        _tc_body,
        grid=(nblk,),
        in_specs=in_specs,
        out_specs=out_specs,
        out_shape=out_shape,
        interpret=interpret,
    )(h1r, adj_r, h2r, aid_r, hmr, tadj_r, *weights)


def _prep_weights(la1_a, mix_a, Wei, bei, Weo, beo, w_ih, w_hh, b_ih, b_hh,
                  b_iah, b_oah):
    w_r, w_i, w_n = w_ih[:DIM], w_ih[DIM:2 * DIM], w_ih[2 * DIM:]
    wh_r, wh_i, wh_n = w_hh[:DIM], w_hh[DIM:2 * DIM], w_hh[2 * DIM:]
    row = lambda v: v.reshape(1, -1)
    return [
        la1_a.T, mix_a.T,
        Wei, row(bei), Weo, row(beo),
        w_r[:, :DIM], w_r[:, DIM:],
        w_i[:, :DIM], w_i[:, DIM:],
        w_n[:, :DIM], w_n[:, DIM:],
        wh_r, wh_i, wh_n,
        row(b_ih[:DIM]), row(b_ih[DIM:2 * DIM]), row(b_ih[2 * DIM:]),
        row(b_hh[:DIM]), row(b_hh[DIM:2 * DIM]), row(b_hh[2 * DIM:]),
        row(b_iah), row(b_oah),
    ]


def kernel(inputs, adj, mask_item, item, items_ID, adj_ID, total_items,
           total_adj, embedding, la1_a, mix_a, Wei, bei, Weo, beo,
           w_ih, w_hh, b_ih, b_hh, b_iah, b_oah):
    emb_p = _pad_table(embedding)
    rows = _make_sc_gather()(
        emb_p, inputs.reshape(-1), items_ID.reshape(-1),
        total_items.reshape(-1))

    weights = _prep_weights(la1_a, mix_a, Wei, bei, Weo, beo, w_ih, w_hh,
                            b_ih, b_hh, b_iah, b_oah)

    o1, o2, om = _tc_call(
        rows, adj.reshape(B * L, L), adj_ID.reshape(B * L, 2 * L),
        total_adj.reshape(B * L, L), weights)

    shp = (B, L, DIM)
    return (o1.reshape(shp), o2.reshape(shp), om.reshape(shp))
